# Initial kernel scaffold; baseline (speedup 1.0000x reference)
import jax
import jax.numpy as jnp
from jax.experimental import pallas as pl


def kernel(input_tensor, grid):
    raise NotImplementedError("write your pallas kernel here")


# speedup vs baseline: 2.1332x; 2.1332x over previous
import functools

import jax
import jax.numpy as jnp
from jax import lax
from jax.experimental import pallas as pl
from jax.experimental.pallas import tpu as pltpu
from jax.experimental.pallas import tpu_sc as plsc

N, C, H, W = 16, 64, 128, 128
HO, WO = 256, 256
P = N * HO * WO
NW = 32
PPW = P // NW
B = 128
NBLK = PPW // B
G = B // 16


def _sc_grid_sample(table, gx, gy):
    mesh = plsc.VectorSubcoreMesh(core_axis_name="c", subcore_axis_name="s")

    @functools.partial(
        pl.kernel,
        out_type=jax.ShapeDtypeStruct((N, C, HO * WO), jnp.float32),
        mesh=mesh,
        scratch_types=[
            pltpu.VMEM((B,), jnp.float32),
            pltpu.VMEM((B,), jnp.float32),
            [pltpu.VMEM((B,), jnp.int32) for _ in range(4)],
            [pltpu.VMEM((B,), jnp.float32) for _ in range(4)],
            [pltpu.VMEM((B, C), jnp.float32) for _ in range(4)],
            pltpu.VMEM((C, B), jnp.float32),
            pltpu.SemaphoreType.DMA,
        ],
        compiler_params=pltpu.CompilerParams(use_tc_tiling_on_sc=False,
                                             needs_layout_passes=False),
    )
    def k(table_hbm, gx_hbm, gy_hbm, out_hbm,
          gxv, gyv, idxs, ws, rows, obuf, sem):
        wid = lax.axis_index("s") * 2 + lax.axis_index("c")
        n = wid // 2
        base_pt = wid * PPW
        img_pt = (wid % 2) * PPW
        rowbase = n * (H * W)

        @pl.loop(0, NBLK)
        def _blk(b):
            p0 = base_pt + b * B
            pltpu.sync_copy(gx_hbm.at[pl.ds(p0, B)], gxv)
            pltpu.sync_copy(gy_hbm.at[pl.ds(p0, B)], gyv)

            for g in range(G):
                sl = pl.ds(g * 16, 16)
                x = gxv[sl]
                y = gyv[sl]
                ix = ((x + 1.0) * float(W) - 1.0) * 0.5
                iy = ((y + 1.0) * float(H) - 1.0) * 0.5
                ixt = ix.astype(jnp.int32).astype(jnp.float32)
                iyt = iy.astype(jnp.int32).astype(jnp.float32)
                ix0 = jnp.where(ixt > ix, ixt - 1.0, ixt)
                iy0 = jnp.where(iyt > iy, iyt - 1.0, iyt)
                ix1 = ix0 + 1.0
                iy1 = iy0 + 1.0
                wx1 = ix - ix0
                wx0 = 1.0 - wx1
                wy1 = iy - iy0
                wy0 = 1.0 - wy1
                vx0 = (ix0 >= 0.0) & (ix0 <= float(W - 1))
                vx1 = (ix1 >= 0.0) & (ix1 <= float(W - 1))
                vy0 = (iy0 >= 0.0) & (iy0 <= float(H - 1))
                vy1 = (iy1 >= 0.0) & (iy1 <= float(H - 1))
                cx0 = jnp.clip(ix0, 0.0, float(W - 1)).astype(jnp.int32)
                cx1 = jnp.clip(ix1, 0.0, float(W - 1)).astype(jnp.int32)
                cy0 = jnp.clip(iy0, 0.0, float(H - 1)).astype(jnp.int32)
                cy1 = jnp.clip(iy1, 0.0, float(H - 1)).astype(jnp.int32)
                idxs[0][sl] = rowbase + cy0 * W + cx0
                idxs[1][sl] = rowbase + cy0 * W + cx1
                idxs[2][sl] = rowbase + cy1 * W + cx0
                idxs[3][sl] = rowbase + cy1 * W + cx1
                ws[0][sl] = jnp.where(vx0 & vy0, wx0 * wy0, 0.0)
                ws[1][sl] = jnp.where(vx1 & vy0, wx1 * wy0, 0.0)
                ws[2][sl] = jnp.where(vx0 & vy1, wx0 * wy1, 0.0)
                ws[3][sl] = jnp.where(vx1 & vy1, wx1 * wy1, 0.0)

            cps = [pltpu.async_copy(table_hbm.at[idxs[j]], rows[j], sem)
                   for j in range(4)]
            for cp in cps:
                cp.wait()

            for g in range(G):
                sl = pl.ds(g * 16, 16)
                w0 = ws[0][sl]
                w1 = ws[1][sl]
                w2 = ws[2][sl]
                w3 = ws[3][sl]
                rix = lax.iota(jnp.int32, 16) + g * 16

                @pl.loop(0, C, unroll=8)
                def _ch(c, w0=w0, w1=w1, w2=w2, w3=w3, rix=rix, sl=sl):
                    col = jnp.full((16,), c, jnp.int32)
                    v0 = plsc.load_gather(rows[0], [rix, col])
                    v1 = plsc.load_gather(rows[1], [rix, col])
                    v2 = plsc.load_gather(rows[2], [rix, col])
                    v3 = plsc.load_gather(rows[3], [rix, col])
                    obuf[c, sl] = v0 * w0 + v1 * w1 + v2 * w2 + v3 * w3

            pltpu.sync_copy(obuf, out_hbm.at[n, :, pl.ds(img_pt + b * B, B)])

    return k(table, gx, gy)


def kernel(input_tensor, grid):
    table = input_tensor.transpose(0, 2, 3, 1).reshape(N * H * W, C)
    gx = grid[..., 0].reshape(P)
    gy = grid[..., 1].reshape(P)
    out = _sc_grid_sample(table, gx, gy)
    return out.reshape(N, C, HO, WO)


# speedup vs baseline: 2.2803x; 1.0690x over previous
import functools

import jax
import jax.numpy as jnp
from jax import lax
from jax.experimental import pallas as pl
from jax.experimental.pallas import tpu as pltpu
from jax.experimental.pallas import tpu_sc as plsc

N, C, H, W = 16, 64, 128, 128
HO, WO = 256, 256
P = N * HO * WO
NW = 32
PPW = P // NW
B = 128
NBLK = PPW // B
G = B // 16


def _sc_grid_sample(table, gx, gy):
    mesh = plsc.VectorSubcoreMesh(core_axis_name="c", subcore_axis_name="s")

    @functools.partial(
        pl.kernel,
        out_type=jax.ShapeDtypeStruct((N, C, HO * WO), jnp.float32),
        mesh=mesh,
        scratch_types=[
            [pltpu.VMEM((B,), jnp.float32) for _ in range(2)],
            [pltpu.VMEM((B,), jnp.float32) for _ in range(2)],
            [[pltpu.VMEM((B,), jnp.int32) for _ in range(4)]
             for _ in range(2)],
            [[pltpu.VMEM((B,), jnp.float32) for _ in range(4)]
             for _ in range(2)],
            [[pltpu.VMEM((B, C), jnp.float32) for _ in range(4)]
             for _ in range(2)],
            [pltpu.VMEM((C, B), jnp.float32) for _ in range(2)],
            [pltpu.SemaphoreType.DMA for _ in range(2)],
            [pltpu.SemaphoreType.DMA for _ in range(2)],
        ],
        compiler_params=pltpu.CompilerParams(use_tc_tiling_on_sc=False,
                                             needs_layout_passes=False),
    )
    def k(table_hbm, gx_hbm, gy_hbm, out_hbm,
          gxv, gyv, idxs, ws, rows, obuf, gsem, osem):
        wid = lax.axis_index("s") * 2 + lax.axis_index("c")
        n = wid // 2
        base_pt = wid * PPW
        img_pt = (wid % 2) * PPW
        rowbase = n * (H * W)

        def stage_block(b, s):
            p0 = base_pt + b * B
            pltpu.sync_copy(gx_hbm.at[pl.ds(p0, B)], gxv[s])
            pltpu.sync_copy(gy_hbm.at[pl.ds(p0, B)], gyv[s])
            for g in range(G):
                sl = pl.ds(g * 16, 16)
                x = gxv[s][sl]
                y = gyv[s][sl]
                ix = ((x + 1.0) * float(W) - 1.0) * 0.5
                iy = ((y + 1.0) * float(H) - 1.0) * 0.5
                ixt = ix.astype(jnp.int32).astype(jnp.float32)
                iyt = iy.astype(jnp.int32).astype(jnp.float32)
                ix0 = jnp.where(ixt > ix, ixt - 1.0, ixt)
                iy0 = jnp.where(iyt > iy, iyt - 1.0, iyt)
                ix1 = ix0 + 1.0
                iy1 = iy0 + 1.0
                wx1 = ix - ix0
                wx0 = 1.0 - wx1
                wy1 = iy - iy0
                wy0 = 1.0 - wy1
                vx0 = (ix0 >= 0.0) & (ix0 <= float(W - 1))
                vx1 = (ix1 >= 0.0) & (ix1 <= float(W - 1))
                vy0 = (iy0 >= 0.0) & (iy0 <= float(H - 1))
                vy1 = (iy1 >= 0.0) & (iy1 <= float(H - 1))
                cx0 = jnp.clip(ix0, 0.0, float(W - 1)).astype(jnp.int32)
                cx1 = jnp.clip(ix1, 0.0, float(W - 1)).astype(jnp.int32)
                cy0 = jnp.clip(iy0, 0.0, float(H - 1)).astype(jnp.int32)
                cy1 = jnp.clip(iy1, 0.0, float(H - 1)).astype(jnp.int32)
                idxs[s][0][sl] = rowbase + cy0 * W + cx0
                idxs[s][1][sl] = rowbase + cy0 * W + cx1
                idxs[s][2][sl] = rowbase + cy1 * W + cx0
                idxs[s][3][sl] = rowbase + cy1 * W + cx1
                ws[s][0][sl] = jnp.where(vx0 & vy0, wx0 * wy0, 0.0)
                ws[s][1][sl] = jnp.where(vx1 & vy0, wx1 * wy0, 0.0)
                ws[s][2][sl] = jnp.where(vx0 & vy1, wx0 * wy1, 0.0)
                ws[s][3][sl] = jnp.where(vx1 & vy1, wx1 * wy1, 0.0)
            for j in range(4):
                pltpu.async_copy(table_hbm.at[idxs[s][j]], rows[s][j], gsem[s])

        def wait_gathers(s):
            for j in range(4):
                pltpu.make_async_copy(table_hbm.at[idxs[s][j]], rows[s][j],
                                      gsem[s]).wait()

        def out_dma(b, s):
            return pltpu.make_async_copy(
                obuf[s], out_hbm.at[n, :, pl.ds(img_pt + b * B, B)], osem[s])

        def combine_block(s):
            zero = jnp.zeros((16,), jnp.int32)
            for half in range(2):
                gs = [half * 4 + g for g in range(4)]
                wv = [[ws[s][j][pl.ds(gg * 16, 16)] for j in range(4)]
                      for gg in gs]
                fidx0 = tuple((lax.iota(jnp.int32, 16) + gg * 16) * C
                              for gg in gs)

                def ch_body(c, fidxs, wv=wv, gs=gs):
                    new = []
                    for t, gg in enumerate(gs):
                        fidx = fidxs[t]
                        v0 = plsc.load_gather(rows[s][0], [zero, fidx])
                        v1 = plsc.load_gather(rows[s][1], [zero, fidx])
                        v2 = plsc.load_gather(rows[s][2], [zero, fidx])
                        v3 = plsc.load_gather(rows[s][3], [zero, fidx])
                        obuf[s][c, pl.ds(gg * 16, 16)] = (
                            v0 * wv[t][0] + v1 * wv[t][1]
                            + v2 * wv[t][2] + v3 * wv[t][3])
                        new.append(fidx + 1)
                    return tuple(new)

                lax.fori_loop(0, C, ch_body, fidx0, unroll=1)

        stage_block(0, 0)

        @pl.loop(0, NBLK // 2)
        def _bb(bb):
            for par in range(2):
                b = bb * 2 + par

                @pl.when(b + 1 < NBLK)
                def _stage():
                    stage_block(b + 1, 1 - par)

                wait_gathers(par)

                @pl.when(b >= 2)
                def _drain():
                    out_dma(b - 2, par).wait()

                combine_block(par)
                out_dma(b, par).start()

        out_dma(NBLK - 2, 0).wait()
        out_dma(NBLK - 1, 1).wait()

    return k(table, gx, gy)


def kernel(input_tensor, grid):
    table = input_tensor.transpose(0, 2, 3, 1).reshape(N * H * W, C)
    gx = grid[..., 0].reshape(P)
    gy = grid[..., 1].reshape(P)
    out = _sc_grid_sample(table, gx, gy)
    return out.reshape(N, C, HO, WO)
